# baseline (speedup 1.0000x reference)
import functools

import jax
import jax.numpy as jnp
from jax import lax
from jax.experimental import pallas as pl
from jax.experimental.pallas import tpu as pltpu
from jax.experimental.pallas import tpu_sc as plsc

_NC = 2
_NS = 16
_NW = _NC * _NS

_B = 4096
_D = 32
_N = 1000000
_BPW = _B // _NW
_CH = 16


def _gather_phase(tbl_hbm, idx16, row0, blocks_v, cols_v, sem):
    qa = (idx16 >> 7) << 7
    ca = idx16 & 127
    d_lo = jnp.arange(16, dtype=jnp.int32)
    d_hi = d_lo + 16
    copies = []
    for j in range(_CH):
        c0 = pl.multiple_of(qa[j], 128)
        copies.append(pltpu.async_copy(
            tbl_hbm.at[:, pl.ds(c0, 128)], blocks_v.at[j], sem))
    for cp in copies:
        cp.wait()
    for j in range(_CH):
        csp = jnp.full((16,), ca[j], jnp.int32)
        isp = jnp.full((16,), row0 + j, jnp.int32)
        lo = plsc.load_gather(blocks_v.at[j], [d_lo, csp])
        hi = plsc.load_gather(blocks_v.at[j], [d_hi, csp])
        plsc.store_scatter(cols_v, [d_lo, isp], lo)
        plsc.store_scatter(cols_v, [d_hi, isp], hi)


def _sc_gather_body(ut_hbm, uidx_hbm, vt_hbm, tidx_hbm, u_out, t_out,
                    uidx_v, tidx_v, blocks_v, ucols_v, tcols_v, sem):
    wid = lax.axis_index("s") * _NC + lax.axis_index("c")
    base = wid * _BPW
    pltpu.sync_copy(uidx_hbm.at[pl.ds(base, _BPW)], uidx_v)
    pltpu.sync_copy(tidx_hbm.at[pl.ds(base, _BPW)], tidx_v)

    def step(g, _):
        o = pl.multiple_of(g * _CH, _CH)
        _gather_phase(ut_hbm, uidx_v[pl.ds(o, 16)], o, blocks_v, ucols_v, sem)
        _gather_phase(vt_hbm, tidx_v[pl.ds(o, 16)], o, blocks_v, tcols_v, sem)
        return 0
    lax.fori_loop(0, _BPW // _CH, step, 0)

    pltpu.sync_copy(ucols_v, u_out.at[:, pl.ds(base, _BPW)])
    pltpu.sync_copy(tcols_v, t_out.at[:, pl.ds(base, _BPW)])


_sc_gather = functools.partial(
    pl.kernel,
    mesh=plsc.VectorSubcoreMesh(core_axis_name="c", subcore_axis_name="s"),
    out_type=[
        jax.ShapeDtypeStruct((_D, _B), jnp.float32),
        jax.ShapeDtypeStruct((_D, _B), jnp.float32),
    ],
    scratch_types=[
        pltpu.VMEM((_BPW,), jnp.int32),
        pltpu.VMEM((_BPW,), jnp.int32),
        pltpu.VMEM((_CH, _D, 128), jnp.float32),
        pltpu.VMEM((_D, _BPW), jnp.float32),
        pltpu.VMEM((_D, _BPW), jnp.float32),
        pltpu.SemaphoreType.DMA,
    ],
    compiler_params=pltpu.CompilerParams(needs_layout_passes=False),
)(_sc_gather_body)


_BM = 1024


def _mm_body(u_ref, t_ref, o_ref):
    o_ref[...] = lax.dot_general(
        u_ref[...], t_ref[...],
        (((0,), (0,)), ((), ())),
        preferred_element_type=jnp.float32,
    )


def kernel(user_indices, item_seq_indices, target_item_indices,
           target_domain, U, V):
    del item_seq_indices, target_domain
    uidx = user_indices.astype(jnp.int32)
    tidx = target_item_indices.reshape(-1).astype(jnp.int32)

    user_ebd_t, tgt_ebd_t = _sc_gather(U.T, uidx, V.T, tidx)

    score = pl.pallas_call(
        _mm_body,
        grid=(_B // _BM,),
        in_specs=[
            pl.BlockSpec((_D, _BM), lambda i: (0, i)),
            pl.BlockSpec((_D, _B), lambda i: (0, 0)),
        ],
        out_specs=pl.BlockSpec((_BM, _B), lambda i: (i, 0)),
        out_shape=jax.ShapeDtypeStruct((_B, _B), jnp.float32),
    )(user_ebd_t, tgt_ebd_t)
    return score


# speedup vs baseline: 1.0210x; 1.0210x over previous
import functools

import jax
import jax.numpy as jnp
from jax import lax
from jax.experimental import pallas as pl
from jax.experimental.pallas import tpu as pltpu
from jax.experimental.pallas import tpu_sc as plsc

_NC = 2
_NS = 16
_NW = _NC * _NS

_B = 4096
_D = 32
_N = 1000000
_BPW = _B // _NW
_CH = 8


def _fire(tbl_hbm, idx16, j0, blocks_v, sem):
    qa = (idx16 >> 7) << 7
    ca = idx16 & 127
    copies = []
    for j in range(j0, j0 + _CH):
        c0 = pl.multiple_of(qa[j], 128)
        copies.append(pltpu.async_copy(
            tbl_hbm.at[:, pl.ds(c0, 128)], blocks_v.at[j - j0], sem))
    return copies, ca


def _extract(copies, ca, j0, row0, blocks_v, cols_v):
    d_lo = jnp.arange(16, dtype=jnp.int32)
    d_hi = d_lo + 16
    for cp in copies:
        cp.wait()
    for j in range(j0, j0 + _CH):
        csp = jnp.full((16,), ca[j], jnp.int32)
        isp = jnp.full((16,), row0 + j, jnp.int32)
        lo = plsc.load_gather(blocks_v.at[j - j0], [d_lo, csp])
        hi = plsc.load_gather(blocks_v.at[j - j0], [d_hi, csp])
        plsc.store_scatter(cols_v, [d_lo, isp], lo)
        plsc.store_scatter(cols_v, [d_hi, isp], hi)


def _sc_gather_body(ut_hbm, uidx_hbm, vt_hbm, tidx_hbm, u_out, t_out,
                    uidx_v, tidx_v, ublocks_v, tblocks_v, ucols_v, tcols_v,
                    sem_u, sem_t):
    wid = lax.axis_index("s") * _NC + lax.axis_index("c")
    base = wid * _BPW
    pltpu.sync_copy(uidx_hbm.at[pl.ds(base, _BPW)], uidx_v)
    pltpu.sync_copy(tidx_hbm.at[pl.ds(base, _BPW)], tidx_v)

    def step(g, _):
        o = pl.multiple_of(g * 16, 16)
        uv = uidx_v[pl.ds(o, 16)]
        tv = tidx_v[pl.ds(o, 16)]
        ucp0, uca0 = _fire(ut_hbm, uv, 0, ublocks_v, sem_u)
        tcp0, tca0 = _fire(vt_hbm, tv, 0, tblocks_v, sem_t)
        _extract(ucp0, uca0, 0, o, ublocks_v, ucols_v)
        ucp1, uca1 = _fire(ut_hbm, uv, _CH, ublocks_v, sem_u)
        _extract(tcp0, tca0, 0, o, tblocks_v, tcols_v)
        tcp1, tca1 = _fire(vt_hbm, tv, _CH, tblocks_v, sem_t)
        _extract(ucp1, uca1, _CH, o, ublocks_v, ucols_v)
        _extract(tcp1, tca1, _CH, o, tblocks_v, tcols_v)
        return 0
    lax.fori_loop(0, _BPW // 16, step, 0)

    pltpu.sync_copy(ucols_v, u_out.at[:, pl.ds(base, _BPW)])
    pltpu.sync_copy(tcols_v, t_out.at[:, pl.ds(base, _BPW)])


_sc_gather = functools.partial(
    pl.kernel,
    mesh=plsc.VectorSubcoreMesh(core_axis_name="c", subcore_axis_name="s"),
    out_type=[
        jax.ShapeDtypeStruct((_D, _B), jnp.float32),
        jax.ShapeDtypeStruct((_D, _B), jnp.float32),
    ],
    scratch_types=[
        pltpu.VMEM((_BPW,), jnp.int32),
        pltpu.VMEM((_BPW,), jnp.int32),
        pltpu.VMEM((_CH, _D, 128), jnp.float32),
        pltpu.VMEM((_CH, _D, 128), jnp.float32),
        pltpu.VMEM((_D, _BPW), jnp.float32),
        pltpu.VMEM((_D, _BPW), jnp.float32),
        pltpu.SemaphoreType.DMA,
        pltpu.SemaphoreType.DMA,
    ],
    compiler_params=pltpu.CompilerParams(needs_layout_passes=False),
)(_sc_gather_body)


_BM = 512


def _mm_body(u_ref, t_ref, o_ref):
    o_ref[...] = lax.dot_general(
        u_ref[...], t_ref[...],
        (((0,), (0,)), ((), ())),
        preferred_element_type=jnp.float32,
    )


def kernel(user_indices, item_seq_indices, target_item_indices,
           target_domain, U, V):
    del item_seq_indices, target_domain
    uidx = user_indices.astype(jnp.int32)
    tidx = target_item_indices.reshape(-1).astype(jnp.int32)

    user_ebd_t, tgt_ebd_t = _sc_gather(U.T, uidx, V.T, tidx)

    score = pl.pallas_call(
        _mm_body,
        grid=(_B // _BM,),
        in_specs=[
            pl.BlockSpec((_D, _BM), lambda i: (0, i)),
            pl.BlockSpec((_D, _B), lambda i: (0, 0)),
        ],
        out_specs=pl.BlockSpec((_BM, _B), lambda i: (i, 0)),
        out_shape=jax.ShapeDtypeStruct((_B, _B), jnp.float32),
    )(user_ebd_t, tgt_ebd_t)
    return score
